# baseline (speedup 1.0000x reference)
import jax
import jax.numpy as jnp
from jax import lax
from jax.experimental import pallas as pl
from jax.experimental.pallas import tpu as pltpu
from jax.experimental.pallas import tpu_sc as plsc

_N = 10000
_E = 320000
_D = 128
_HALF = 64
_HID = 512
_ALPHA = 0.5

_ROWS = _E // 128
_TILES = 16
_RPT = 156
_CHUNK_ROWS = 3
_CHUNK = _CHUNK_ROWS * 128
_NCH = _RPT // _CHUNK_ROWS
_NPAIR = _NCH // 2
_ROWS_PER_TILE = _N // _TILES


def _sc_layer_body(x_lo, x_hi, idx3d, ew2d,
                   f1lo, f1hi, f2lo, f2hi, f3lo, f3hi,
                   idxb, ewb, rows, acc,
                   isem0, isem1, gsem0, gsem1, ssem0, ssem1):
    c = lax.axis_index("c")
    s = lax.axis_index("s")
    f32 = jnp.float32

    isems = (isem0, isem1)
    gsems = (gsem0, gsem1)
    ssems = (ssem0, ssem1)

    row0 = _RPT * s + jnp.minimum(s, 4)
    arow0 = s * _ROWS_PER_TILE
    zeros16 = jnp.zeros((16,), f32)

    def zero_acc():
        def zero_body(i, _):
            for d in range(4):
                rows[0, i, pl.ds(d * 16, 16)] = zeros16
            return 0

        lax.fori_loop(0, _CHUNK, zero_body, 0)
        pltpu.sync_copy(rows.at[0], acc.at[pl.ds(arow0, _CHUNK)])
        pltpu.sync_copy(rows.at[0].at[pl.ds(0, _ROWS_PER_TILE - _CHUNK)],
                        acc.at[pl.ds(arow0 + _CHUNK,
                                     _ROWS_PER_TILE - _CHUNK)])

    def fire_idx(b, k):
        g = row0 + 3 * k
        pltpu.async_copy(idx3d.at[pl.ds(g, _CHUNK_ROWS)], idxb.at[b],
                         isems[b])
        pltpu.async_copy(ew2d.at[pl.ds(g, _CHUNK_ROWS)], ewb.at[b],
                         isems[b])

    def wait_idx(b, k):
        g = row0 + 3 * k
        pltpu.make_async_copy(idx3d.at[pl.ds(g, _CHUNK_ROWS)], idxb.at[b],
                              isems[b]).wait()
        pltpu.make_async_copy(ew2d.at[pl.ds(g, _CHUNK_ROWS)], ewb.at[b],
                              isems[b]).wait()

    def scale(b, k):
        for j in range(_CHUNK_ROWS):
            def group(i, _):
                wv = ewb[b, j, pl.ds(i * 16, 16)]
                for u in range(16):
                    w = wv[u]
                    e = j * 128 + i * 16 + u
                    for d in range(4):
                        sl = pl.ds(d * 16, 16)
                        rows[b, e, sl] = rows[b, e, sl] * w
                return 0

            lax.fori_loop(0, 8, group, 0)

    def run_layer(h_half, out_half):
        def fire_gather(b, k):
            for j in range(_CHUNK_ROWS):
                pltpu.async_copy(h_half.at[idxb.at[b, j, 0]],
                                 rows.at[b].at[pl.ds(j * 128, 128)],
                                 gsems[b])

        def wait_gather(b, k):
            for j in range(_CHUNK_ROWS):
                pltpu.make_async_copy(h_half.at[idxb.at[b, j, 0]],
                                      rows.at[b].at[pl.ds(j * 128, 128)],
                                      gsems[b]).wait()

        def fire_scatter(b, k):
            for j in range(_CHUNK_ROWS):
                pltpu.async_copy(rows.at[b].at[pl.ds(j * 128, 128)],
                                 acc.at[idxb.at[b, j, 1]],
                                 ssems[b], add=True)

        def wait_scatter(b, k):
            for j in range(_CHUNK_ROWS):
                pltpu.make_async_copy(rows.at[b].at[pl.ds(j * 128, 128)],
                                      acc.at[idxb.at[b, j, 1]],
                                      ssems[b]).wait()

        fire_idx(0, jnp.int32(0))
        wait_idx(0, jnp.int32(0))
        fire_gather(0, jnp.int32(0))

        def pair(t, _):
            ka = 2 * t
            kb = 2 * t + 1

            @pl.when(t > 0)
            def _():
                wait_scatter(1, kb - 2)

            fire_idx(1, kb)
            wait_gather(0, ka)
            wait_idx(1, kb)
            fire_gather(1, kb)
            scale(0, ka)
            fire_scatter(0, ka)
            wait_gather(1, kb)

            @pl.when(t < _NPAIR - 1)
            def _():
                wait_scatter(0, ka)
                fire_idx(0, ka + 2)
                wait_idx(0, ka + 2)
                fire_gather(0, ka + 2)

            scale(1, kb)
            fire_scatter(1, kb)
            return 0

        lax.fori_loop(0, _NPAIR, pair, 0)
        wait_scatter(0, jnp.int32(_NCH - 2))
        wait_scatter(1, jnp.int32(_NCH - 1))

        @pl.when(s < 4)
        def _():
            g = row0 + _RPT
            pltpu.sync_copy(idx3d.at[pl.ds(g, 1)],
                            idxb.at[0].at[pl.ds(0, 1)])
            pltpu.sync_copy(ew2d.at[pl.ds(g, 1)],
                            ewb.at[0].at[pl.ds(0, 1)])
            pltpu.sync_copy(h_half.at[idxb.at[0, 0, 0]],
                            rows.at[0].at[pl.ds(0, 128)])

            def group(i, _):
                wv = ewb[0, 0, pl.ds(i * 16, 16)]
                for u in range(16):
                    w = wv[u]
                    e = i * 16 + u
                    for d in range(4):
                        sl = pl.ds(d * 16, 16)
                        rows[0, e, sl] = rows[0, e, sl] * w
                return 0

            lax.fori_loop(0, 8, group, 0)
            pltpu.sync_copy(rows.at[0].at[pl.ds(0, 128)],
                            acc.at[idxb.at[0, 0, 1]], add=True)

        plsc.subcore_barrier()
        pltpu.sync_copy(acc.at[pl.ds(arow0, _CHUNK)], rows.at[0])
        pltpu.sync_copy(acc.at[pl.ds(arow0 + _CHUNK,
                                     _ROWS_PER_TILE - _CHUNK)],
                        rows.at[1].at[pl.ds(0, _ROWS_PER_TILE - _CHUNK)])
        pltpu.sync_copy(rows.at[0], out_half.at[pl.ds(arow0, _CHUNK)])
        pltpu.sync_copy(rows.at[1].at[pl.ds(0, _ROWS_PER_TILE - _CHUNK)],
                        out_half.at[pl.ds(arow0 + _CHUNK,
                                          _ROWS_PER_TILE - _CHUNK)])
        plsc.subcore_barrier()

    def run_all(hops):
        for l in range(3):
            zero_acc()
            plsc.subcore_barrier()
            run_layer(hops[l], hops[l + 1])

    @pl.when(c == 0)
    def _():
        run_all([x_lo, f1lo, f2lo, f3lo])

    @pl.when(c == 1)
    def _():
        run_all([x_hi, f1hi, f2hi, f3hi])


def _sc_layers(x_lo, x_hi, idx3d, ew2d):
    mesh = plsc.VectorSubcoreMesh(core_axis_name="c", subcore_axis_name="s")
    f32 = jnp.float32
    hh = jax.ShapeDtypeStruct((_N, _HALF), f32)
    return pl.kernel(
        _sc_layer_body,
        out_type=[hh, hh, hh, hh, hh, hh],
        mesh=mesh,
        scratch_types=[
            pltpu.VMEM((2, _CHUNK_ROWS, 2, 128), jnp.int32),
            pltpu.VMEM((2, _CHUNK_ROWS, 128), f32),
            pltpu.VMEM((2, _CHUNK, _HALF), f32),
            pltpu.VMEM_SHARED((_N, _HALF), f32),
            pltpu.SemaphoreType.DMA,
            pltpu.SemaphoreType.DMA,
            pltpu.SemaphoreType.DMA,
            pltpu.SemaphoreType.DMA,
            pltpu.SemaphoreType.DMA,
            pltpu.SemaphoreType.DMA,
        ],
        compiler_params=pltpu.CompilerParams(use_tc_tiling_on_sc=False),
        name="gamlp_sc_layers",
    )(x_lo, x_hi, idx3d, ew2d)


def _head_body(x_ref, f1l, f1h, f2l, f2h, f3l, f3h,
               wat, batt, w0, wg1, wg2, w3, pa, o_ref):
    f32 = jnp.float32
    f0 = x_ref[...]
    f1 = jnp.concatenate([f1l[...], f1h[...]], axis=1)
    f2 = jnp.concatenate([f2l[...], f2h[...]], axis=1)
    f3 = jnp.concatenate([f3l[...], f3h[...]], axis=1)

    wa0 = wat[0:_D, :]
    wa1 = wat[_D:2 * _D, :]
    b = batt[0, 0]

    def mv(f, w):
        return jnp.dot(f, w, preferred_element_type=f32)

    g0 = mv(f0, wa0)
    g1 = mv(f1, wa0)
    g2 = mv(f2, wa0)
    e0 = mv(f0, wa1)
    e1 = mv(f1, wa1)
    e2 = mv(f2, wa1)
    e3 = mv(f3, wa1)

    s0 = jnp.tanh(g0 + e0 + b)
    s1 = jnp.tanh(g0 + e1 + b)
    a = jax.nn.softmax(jnp.concatenate([s0, s1], axis=1), axis=1)
    s2 = jnp.tanh(a[:, 0:1] * g0 + a[:, 1:2] * g1 + e2 + b)
    a = jax.nn.softmax(jnp.concatenate([s0, s1, s2], axis=1), axis=1)
    s3 = jnp.tanh(a[:, 0:1] * g0 + a[:, 1:2] * g1 + a[:, 2:3] * g2 + e3 + b)

    a = jax.nn.softmax(jnp.concatenate([s0, s1, s2, s3], axis=1), axis=1)
    right = (f0 * a[:, 0:1] + f1 * a[:, 1:2]
             + f2 * a[:, 2:3] + f3 * a[:, 3:4])

    alpha = pa[0, 0]

    def prelu(v):
        return jnp.where(v >= 0, v, alpha * v)

    z = jnp.dot(right, w0[...], preferred_element_type=f32)
    h0 = z
    z = prelu(z)
    z = jnp.dot((1.0 - _ALPHA) * z + _ALPHA * h0, wg1[...],
                preferred_element_type=f32)
    z = prelu(z)
    z = jnp.dot((1.0 - _ALPHA) * z + _ALPHA * h0, wg2[...],
                preferred_element_type=f32)
    z = prelu(z)
    o_ref[...] = jnp.dot(z, w3[...], preferred_element_type=f32)


def _head(x, f1l, f1h, f2l, f2h, f3l, f3h, W_att, b_att,
          W0, Wg1, Wg2, W3, prelu_a):
    f32 = jnp.float32
    BN = 1000
    grid = (_N // BN,)

    def row_spec(w):
        return pl.BlockSpec((BN, w), lambda i: (i, 0))

    def full_spec(shape):
        return pl.BlockSpec(shape, lambda i: tuple(0 for _ in shape))

    return pl.pallas_call(
        _head_body,
        grid=grid,
        in_specs=[
            row_spec(_D),
            row_spec(_HALF), row_spec(_HALF),
            row_spec(_HALF), row_spec(_HALF),
            row_spec(_HALF), row_spec(_HALF),
            full_spec((2 * _D, 1)),
            full_spec((1, 1)),
            full_spec((_D, _HID)),
            full_spec((_HID, _HID)),
            full_spec((_HID, _HID)),
            full_spec((_HID, _D)),
            full_spec((1, 1)),
        ],
        out_specs=row_spec(_D),
        out_shape=jax.ShapeDtypeStruct((_N, _D), f32),
    )(x, f1l, f1h, f2l, f2h, f3l, f3h, W_att, b_att.reshape(1, 1),
      W0, Wg1, Wg2, W3, prelu_a.reshape(1, 1))


def kernel(x, edge_index, edge_weight, W_att, b_att, W0, Wg1, Wg2, W3,
           prelu_a):
    src2d = edge_index[0].astype(jnp.int32).reshape(_ROWS, 128)
    dst2d = edge_index[1].astype(jnp.int32).reshape(_ROWS, 128)
    ew2d = edge_weight.reshape(_ROWS, 128)
    idx3d = jnp.stack([src2d, dst2d], axis=1)

    x_lo = x[:, :_HALF]
    x_hi = x[:, _HALF:]

    f1l, f1h, f2l, f2h, f3l, f3h = _sc_layers(x_lo, x_hi, idx3d, ew2d)

    return _head(x, f1l, f1h, f2l, f2h, f3l, f3h,
                 W_att, b_att, W0, Wg1, Wg2, W3, prelu_a)


# speedup vs baseline: 1.1339x; 1.1339x over previous
import jax
import jax.numpy as jnp
from jax import lax
from jax.experimental import pallas as pl
from jax.experimental.pallas import tpu as pltpu
from jax.experimental.pallas import tpu_sc as plsc

_N = 10000
_E = 320000
_D = 128
_HALF = 64
_HID = 512
_ALPHA = 0.5

_ROWS = _E // 128
_TILES = 16
_RPT = 156
_CHUNK_ROWS = 2
_CHUNK = _CHUNK_ROWS * 128
_NCH = _RPT // _CHUNK_ROWS
_NRB = 3
_ROWS_PER_TILE = _N // _TILES


def _sc_layer_body(x_lo, x_hi, idx3d, ew2d,
                   f1lo, f1hi, f2lo, f2hi, f3lo, f3hi,
                   idxb, ewb, dstb, rows, acc,
                   isem0, isem1, isem2,
                   gsem0, gsem1, gsem2, ssem0, ssem1, ssem2):
    c = lax.axis_index("c")
    s = lax.axis_index("s")
    f32 = jnp.float32

    isems = (isem0, isem1, isem2)
    gsems = (gsem0, gsem1, gsem2)
    ssems = (ssem0, ssem1, ssem2)

    row0 = _RPT * s + jnp.minimum(s, 4)
    arow0 = s * _ROWS_PER_TILE
    zeros16 = jnp.zeros((16,), f32)

    def zero_acc():
        def zero_body(i, _):
            for d in range(4):
                rows[0, i, pl.ds(d * 16, 16)] = zeros16
            return 0

        lax.fori_loop(0, _CHUNK, zero_body, 0)
        rem = _ROWS_PER_TILE - 2 * _CHUNK
        pltpu.sync_copy(rows.at[0], acc.at[pl.ds(arow0, _CHUNK)])
        pltpu.sync_copy(rows.at[0], acc.at[pl.ds(arow0 + _CHUNK, _CHUNK)])
        pltpu.sync_copy(rows.at[0].at[pl.ds(0, rem)],
                        acc.at[pl.ds(arow0 + 2 * _CHUNK, rem)])

    def fire_idx(q, k):
        g = row0 + _CHUNK_ROWS * k
        pltpu.async_copy(idx3d.at[pl.ds(g, _CHUNK_ROWS)], idxb.at[q],
                         isems[q])
        pltpu.async_copy(ew2d.at[pl.ds(g, _CHUNK_ROWS)], ewb.at[q],
                         isems[q])

    def wait_idx(q, k):
        g = row0 + _CHUNK_ROWS * k
        pltpu.make_async_copy(idx3d.at[pl.ds(g, _CHUNK_ROWS)], idxb.at[q],
                              isems[q]).wait()
        pltpu.make_async_copy(ew2d.at[pl.ds(g, _CHUNK_ROWS)], ewb.at[q],
                              isems[q]).wait()

    def scale(r, q):
        for j in range(_CHUNK_ROWS):
            def group(i, _):
                wv = ewb[q, j, pl.ds(i * 16, 16)]
                for u in range(16):
                    w = wv[u]
                    e = j * 128 + i * 16 + u
                    for d in range(4):
                        sl = pl.ds(d * 16, 16)
                        rows[r, e, sl] = rows[r, e, sl] * w
                return 0

            lax.fori_loop(0, 8, group, 0)

    def run_layer(h_half, out_half):
        def fire_gather(r, k):
            for j in range(_CHUNK_ROWS):
                pltpu.async_copy(h_half.at[idxb.at[r, j, 0]],
                                 rows.at[r].at[pl.ds(j * 128, 128)],
                                 gsems[r])

        def wait_gather(r, k):
            for j in range(_CHUNK_ROWS):
                pltpu.make_async_copy(h_half.at[idxb.at[r, j, 0]],
                                      rows.at[r].at[pl.ds(j * 128, 128)],
                                      gsems[r]).wait()

        def copy_dst(r):
            for j in range(_CHUNK_ROWS):
                for i in range(8):
                    sl = pl.ds(i * 16, 16)
                    dstb[r, j, sl] = idxb[r, j, 1, sl]

        def fire_scatter(r, k):
            for j in range(_CHUNK_ROWS):
                pltpu.async_copy(rows.at[r].at[pl.ds(j * 128, 128)],
                                 acc.at[dstb.at[r, j]],
                                 ssems[r], add=True)

        def wait_scatter(r, k):
            for j in range(_CHUNK_ROWS):
                pltpu.make_async_copy(rows.at[r].at[pl.ds(j * 128, 128)],
                                      acc.at[dstb.at[r, j]],
                                      ssems[r]).wait()

        def chunk_ops(k, r):
            r1 = (r + 1) % _NRB
            r2 = (r + 2) % _NRB

            @pl.when(k >= 2)
            def _():
                wait_scatter(r1, k - 2)

            wait_gather(r, k)
            copy_dst(r)

            @pl.when(k + 2 < _NCH)
            def _():
                fire_idx(r2, k + 2)

            @pl.when(k + 1 < _NCH)
            def _():
                wait_idx(r1, k + 1)
                fire_gather(r1, k + 1)

            scale(r, r)
            fire_scatter(r, k)

        fire_idx(0, jnp.int32(0))
        fire_idx(1, jnp.int32(1))
        wait_idx(0, jnp.int32(0))
        fire_gather(0, jnp.int32(0))

        def triple(t, _):
            for j in range(3):
                chunk_ops(3 * t + j, j)
            return 0

        lax.fori_loop(0, _NCH // 3, triple, 0)
        wait_scatter((_NCH - 2) % _NRB, jnp.int32(_NCH - 2))
        wait_scatter((_NCH - 1) % _NRB, jnp.int32(_NCH - 1))

        @pl.when(s < 4)
        def _():
            g = row0 + _RPT
            pltpu.sync_copy(idx3d.at[pl.ds(g, 1)],
                            idxb.at[0].at[pl.ds(0, 1)])
            pltpu.sync_copy(ew2d.at[pl.ds(g, 1)],
                            ewb.at[0].at[pl.ds(0, 1)])
            pltpu.sync_copy(h_half.at[idxb.at[0, 0, 0]],
                            rows.at[0].at[pl.ds(0, 128)])

            def group(i, _):
                wv = ewb[0, 0, pl.ds(i * 16, 16)]
                for u in range(16):
                    w = wv[u]
                    e = i * 16 + u
                    for d in range(4):
                        sl = pl.ds(d * 16, 16)
                        rows[0, e, sl] = rows[0, e, sl] * w
                return 0

            lax.fori_loop(0, 8, group, 0)
            pltpu.sync_copy(rows.at[0].at[pl.ds(0, 128)],
                            acc.at[idxb.at[0, 0, 1]], add=True)

        plsc.subcore_barrier()
        rem = _ROWS_PER_TILE - 2 * _CHUNK
        pltpu.sync_copy(acc.at[pl.ds(arow0, _CHUNK)], rows.at[0])
        pltpu.sync_copy(acc.at[pl.ds(arow0 + _CHUNK, _CHUNK)], rows.at[1])
        pltpu.sync_copy(acc.at[pl.ds(arow0 + 2 * _CHUNK, rem)],
                        rows.at[2].at[pl.ds(0, rem)])
        pltpu.sync_copy(rows.at[0], out_half.at[pl.ds(arow0, _CHUNK)])
        pltpu.sync_copy(rows.at[1],
                        out_half.at[pl.ds(arow0 + _CHUNK, _CHUNK)])
        pltpu.sync_copy(rows.at[2].at[pl.ds(0, rem)],
                        out_half.at[pl.ds(arow0 + 2 * _CHUNK, rem)])
        plsc.subcore_barrier()

    def run_all(hops):
        for l in range(3):
            zero_acc()
            plsc.subcore_barrier()
            run_layer(hops[l], hops[l + 1])

    @pl.when(c == 0)
    def _():
        run_all([x_lo, f1lo, f2lo, f3lo])

    @pl.when(c == 1)
    def _():
        run_all([x_hi, f1hi, f2hi, f3hi])


def _sc_layers(x_lo, x_hi, idx3d, ew2d):
    mesh = plsc.VectorSubcoreMesh(core_axis_name="c", subcore_axis_name="s")
    f32 = jnp.float32
    hh = jax.ShapeDtypeStruct((_N, _HALF), f32)
    return pl.kernel(
        _sc_layer_body,
        out_type=[hh, hh, hh, hh, hh, hh],
        mesh=mesh,
        scratch_types=[
            pltpu.VMEM((_NRB, _CHUNK_ROWS, 2, 128), jnp.int32),
            pltpu.VMEM((_NRB, _CHUNK_ROWS, 128), f32),
            pltpu.VMEM((_NRB, _CHUNK_ROWS, 128), jnp.int32),
            pltpu.VMEM((_NRB, _CHUNK, _HALF), f32),
            pltpu.VMEM_SHARED((_N, _HALF), f32),
        ] + [pltpu.SemaphoreType.DMA] * 9,
        compiler_params=pltpu.CompilerParams(use_tc_tiling_on_sc=False),
        name="gamlp_sc_layers",
    )(x_lo, x_hi, idx3d, ew2d)


def _head_body(x_ref, f1l, f1h, f2l, f2h, f3l, f3h,
               wat, batt, w0, wg1, wg2, w3, pa, o_ref):
    f32 = jnp.float32
    f0 = x_ref[...]
    f1 = jnp.concatenate([f1l[...], f1h[...]], axis=1)
    f2 = jnp.concatenate([f2l[...], f2h[...]], axis=1)
    f3 = jnp.concatenate([f3l[...], f3h[...]], axis=1)

    wa0 = wat[0:_D, :]
    wa1 = wat[_D:2 * _D, :]
    b = batt[0, 0]

    def mv(f, w):
        return jnp.dot(f, w, preferred_element_type=f32)

    g0 = mv(f0, wa0)
    g1 = mv(f1, wa0)
    g2 = mv(f2, wa0)
    e0 = mv(f0, wa1)
    e1 = mv(f1, wa1)
    e2 = mv(f2, wa1)
    e3 = mv(f3, wa1)

    s0 = jnp.tanh(g0 + e0 + b)
    s1 = jnp.tanh(g0 + e1 + b)
    a = jax.nn.softmax(jnp.concatenate([s0, s1], axis=1), axis=1)
    s2 = jnp.tanh(a[:, 0:1] * g0 + a[:, 1:2] * g1 + e2 + b)
    a = jax.nn.softmax(jnp.concatenate([s0, s1, s2], axis=1), axis=1)
    s3 = jnp.tanh(a[:, 0:1] * g0 + a[:, 1:2] * g1 + a[:, 2:3] * g2 + e3 + b)

    a = jax.nn.softmax(jnp.concatenate([s0, s1, s2, s3], axis=1), axis=1)
    right = (f0 * a[:, 0:1] + f1 * a[:, 1:2]
             + f2 * a[:, 2:3] + f3 * a[:, 3:4])

    alpha = pa[0, 0]

    def prelu(v):
        return jnp.where(v >= 0, v, alpha * v)

    z = jnp.dot(right, w0[...], preferred_element_type=f32)
    h0 = z
    z = prelu(z)
    z = jnp.dot((1.0 - _ALPHA) * z + _ALPHA * h0, wg1[...],
                preferred_element_type=f32)
    z = prelu(z)
    z = jnp.dot((1.0 - _ALPHA) * z + _ALPHA * h0, wg2[...],
                preferred_element_type=f32)
    z = prelu(z)
    o_ref[...] = jnp.dot(z, w3[...], preferred_element_type=f32)


def _head(x, f1l, f1h, f2l, f2h, f3l, f3h, W_att, b_att,
          W0, Wg1, Wg2, W3, prelu_a):
    f32 = jnp.float32
    BN = 1000
    grid = (_N // BN,)

    def row_spec(w):
        return pl.BlockSpec((BN, w), lambda i: (i, 0))

    def full_spec(shape):
        return pl.BlockSpec(shape, lambda i: tuple(0 for _ in shape))

    return pl.pallas_call(
        _head_body,
        grid=grid,
        in_specs=[
            row_spec(_D),
            row_spec(_HALF), row_spec(_HALF),
            row_spec(_HALF), row_spec(_HALF),
            row_spec(_HALF), row_spec(_HALF),
            full_spec((2 * _D, 1)),
            full_spec((1, 1)),
            full_spec((_D, _HID)),
            full_spec((_HID, _HID)),
            full_spec((_HID, _HID)),
            full_spec((_HID, _D)),
            full_spec((1, 1)),
        ],
        out_specs=row_spec(_D),
        out_shape=jax.ShapeDtypeStruct((_N, _D), f32),
    )(x, f1l, f1h, f2l, f2h, f3l, f3h, W_att, b_att.reshape(1, 1),
      W0, Wg1, Wg2, W3, prelu_a.reshape(1, 1))


def kernel(x, edge_index, edge_weight, W_att, b_att, W0, Wg1, Wg2, W3,
           prelu_a):
    src2d = edge_index[0].astype(jnp.int32).reshape(_ROWS, 128)
    dst2d = edge_index[1].astype(jnp.int32).reshape(_ROWS, 128)
    ew2d = edge_weight.reshape(_ROWS, 128)
    idx3d = jnp.stack([src2d, dst2d], axis=1)

    x_lo = x[:, :_HALF]
    x_hi = x[:, _HALF:]

    f1l, f1h, f2l, f2h, f3l, f3h = _sc_layers(x_lo, x_hi, idx3d, ew2d)

    return _head(x, f1l, f1h, f2l, f2h, f3l, f3h,
                 W_att, b_att, W0, Wg1, Wg2, W3, prelu_a)


# speedup vs baseline: 1.1956x; 1.0544x over previous
import jax
import jax.numpy as jnp
from jax import lax
from jax.experimental import pallas as pl
from jax.experimental.pallas import tpu as pltpu
from jax.experimental.pallas import tpu_sc as plsc

_N = 10000
_E = 320000
_D = 128
_HALF = 64
_HID = 512
_ALPHA = 0.5

_ROWS = _E // 128
_TILES = 16
_RPT = 156
_CHUNK_ROWS = 2
_CHUNK = _CHUNK_ROWS * 128
_NCH = _RPT // _CHUNK_ROWS
_NRB = 3
_ROWS_PER_TILE = _N // _TILES


def _sc_layer_body(x_lo, x_hi, idx3d, ew2d,
                   f1lo, f1hi, f2lo, f2hi, f3lo, f3hi,
                   idxb, ewb, dstb, rows, acc,
                   isem0, isem1, isem2,
                   gsem0, gsem1, gsem2, ssem0, ssem1, ssem2):
    c = lax.axis_index("c")
    s = lax.axis_index("s")
    f32 = jnp.float32

    isems = (isem0, isem1, isem2)
    gsems = (gsem0, gsem1, gsem2)
    ssems = (ssem0, ssem1, ssem2)

    row0 = _RPT * s + jnp.minimum(s, 4)
    arow0 = s * _ROWS_PER_TILE
    zeros16 = jnp.zeros((16,), f32)

    def zero_acc():
        def zero_body(i, _):
            for d in range(4):
                rows[0, i, pl.ds(d * 16, 16)] = zeros16
            return 0

        lax.fori_loop(0, _CHUNK, zero_body, 0)
        rem = _ROWS_PER_TILE - 2 * _CHUNK
        pltpu.sync_copy(rows.at[0], acc.at[pl.ds(arow0, _CHUNK)])
        pltpu.sync_copy(rows.at[0], acc.at[pl.ds(arow0 + _CHUNK, _CHUNK)])
        pltpu.sync_copy(rows.at[0].at[pl.ds(0, rem)],
                        acc.at[pl.ds(arow0 + 2 * _CHUNK, rem)])

    def fire_idx(q, k):
        g = row0 + _CHUNK_ROWS * k
        pltpu.async_copy(idx3d.at[pl.ds(g, _CHUNK_ROWS)], idxb.at[q],
                         isems[q])
        pltpu.async_copy(ew2d.at[pl.ds(g, _CHUNK_ROWS)], ewb.at[q],
                         isems[q])

    def wait_idx(q, k):
        g = row0 + _CHUNK_ROWS * k
        pltpu.make_async_copy(idx3d.at[pl.ds(g, _CHUNK_ROWS)], idxb.at[q],
                              isems[q]).wait()
        pltpu.make_async_copy(ew2d.at[pl.ds(g, _CHUNK_ROWS)], ewb.at[q],
                              isems[q]).wait()

    def scale(r, q):
        for j in range(_CHUNK_ROWS):
            def group(i, _):
                wv = ewb[q, j, pl.ds(i * 16, 16)]
                for u in range(16):
                    w = wv[u]
                    e = j * 128 + i * 16 + u
                    for d in range(4):
                        sl = pl.ds(d * 16, 16)
                        rows[r, e, sl] = rows[r, e, sl] * w
                return 0

            lax.fori_loop(0, 8, group, 0)

    def run_layer(h_half, out_half):
        def fire_gather(r, k):
            for j in range(_CHUNK_ROWS):
                pltpu.async_copy(h_half.at[idxb.at[r, j, 0]],
                                 rows.at[r].at[pl.ds(j * 128, 128)],
                                 gsems[r])

        def wait_gather(r, k):
            for j in range(_CHUNK_ROWS):
                pltpu.make_async_copy(h_half.at[idxb.at[r, j, 0]],
                                      rows.at[r].at[pl.ds(j * 128, 128)],
                                      gsems[r]).wait()

        def copy_dst(r):
            for j in range(_CHUNK_ROWS):
                for i in range(8):
                    sl = pl.ds(i * 16, 16)
                    dstb[r, j, sl] = idxb[r, j, 1, sl]

        def fire_scatter(r, k):
            for j in range(_CHUNK_ROWS):
                pltpu.async_copy(rows.at[r].at[pl.ds(j * 128, 128)],
                                 acc.at[dstb.at[r, j]],
                                 ssems[r], add=True)

        def wait_scatter(r, k):
            for j in range(_CHUNK_ROWS):
                pltpu.make_async_copy(rows.at[r].at[pl.ds(j * 128, 128)],
                                      acc.at[dstb.at[r, j]],
                                      ssems[r]).wait()

        def chunk_ops(k, r):
            r1 = (r + 1) % _NRB
            r2 = (r + 2) % _NRB

            @pl.when(k >= 2)
            def _():
                wait_scatter(r1, k - 2)

            wait_gather(r, k)
            copy_dst(r)

            @pl.when(k + 2 < _NCH)
            def _():
                fire_idx(r2, k + 2)

            @pl.when(k + 1 < _NCH)
            def _():
                wait_idx(r1, k + 1)
                fire_gather(r1, k + 1)

            scale(r, r)
            fire_scatter(r, k)

        fire_idx(0, jnp.int32(0))
        fire_idx(1, jnp.int32(1))
        wait_idx(0, jnp.int32(0))
        fire_gather(0, jnp.int32(0))

        def triple(t, _):
            for j in range(3):
                chunk_ops(3 * t + j, j)
            return 0

        lax.fori_loop(0, _NCH // 3, triple, 0)
        wait_scatter((_NCH - 2) % _NRB, jnp.int32(_NCH - 2))
        wait_scatter((_NCH - 1) % _NRB, jnp.int32(_NCH - 1))

        @pl.when(s < 4)
        def _():
            g = row0 + _RPT
            pltpu.sync_copy(idx3d.at[pl.ds(g, 1)],
                            idxb.at[0].at[pl.ds(0, 1)])
            pltpu.sync_copy(ew2d.at[pl.ds(g, 1)],
                            ewb.at[0].at[pl.ds(0, 1)])
            pltpu.sync_copy(h_half.at[idxb.at[0, 0, 0]],
                            rows.at[0].at[pl.ds(0, 128)])

            def group(i, _):
                wv = ewb[0, 0, pl.ds(i * 16, 16)]
                for u in range(16):
                    w = wv[u]
                    e = i * 16 + u
                    for d in range(4):
                        sl = pl.ds(d * 16, 16)
                        rows[0, e, sl] = rows[0, e, sl] * w
                return 0

            lax.fori_loop(0, 8, group, 0)
            pltpu.sync_copy(rows.at[0].at[pl.ds(0, 128)],
                            acc.at[idxb.at[0, 0, 1]], add=True)

        plsc.subcore_barrier()
        rem = _ROWS_PER_TILE - 2 * _CHUNK
        pltpu.sync_copy(acc.at[pl.ds(arow0, _CHUNK)], rows.at[0])
        pltpu.sync_copy(acc.at[pl.ds(arow0 + _CHUNK, _CHUNK)], rows.at[1])
        pltpu.sync_copy(acc.at[pl.ds(arow0 + 2 * _CHUNK, rem)],
                        rows.at[2].at[pl.ds(0, rem)])
        pltpu.sync_copy(rows.at[0], out_half.at[pl.ds(arow0, _CHUNK)])
        pltpu.sync_copy(rows.at[1],
                        out_half.at[pl.ds(arow0 + _CHUNK, _CHUNK)])
        pltpu.sync_copy(rows.at[2].at[pl.ds(0, rem)],
                        out_half.at[pl.ds(arow0 + 2 * _CHUNK, rem)])
        plsc.subcore_barrier()

    def run_all(hops):
        for l in range(3):
            zero_acc()
            plsc.subcore_barrier()
            run_layer(hops[l], hops[l + 1])

    @pl.when(c == 0)
    def _():
        run_all([x_lo, f1lo, f2lo, f3lo])

    @pl.when(c == 1)
    def _():
        run_all([x_hi, f1hi, f2hi, f3hi])


def _sc_layers(x_lo, x_hi, idx3d, ew2d):
    mesh = plsc.VectorSubcoreMesh(core_axis_name="c", subcore_axis_name="s")
    f32 = jnp.float32
    hh = jax.ShapeDtypeStruct((_N, _HALF), f32)
    return pl.kernel(
        _sc_layer_body,
        out_type=[hh, hh, hh, hh, hh, hh],
        mesh=mesh,
        scratch_types=[
            pltpu.VMEM((_NRB, _CHUNK_ROWS, 2, 128), jnp.int32),
            pltpu.VMEM((_NRB, _CHUNK_ROWS, 128), f32),
            pltpu.VMEM((_NRB, _CHUNK_ROWS, 128), jnp.int32),
            pltpu.VMEM((_NRB, _CHUNK, _HALF), f32),
            pltpu.VMEM_SHARED((_N, _HALF), f32),
        ] + [pltpu.SemaphoreType.DMA] * 9,
        compiler_params=pltpu.CompilerParams(use_tc_tiling_on_sc=False),
        name="gamlp_sc_layers",
    )(x_lo, x_hi, idx3d, ew2d)


def _head_body(x_ref, f1l, f1h, f2l, f2h, f3l, f3h,
               wat, batt, w0, wg1, wg2, w3, pa, o_ref):
    f32 = jnp.float32
    f0 = x_ref[...]
    f1 = jnp.concatenate([f1l[...], f1h[...]], axis=1)
    f2 = jnp.concatenate([f2l[...], f2h[...]], axis=1)
    f3 = jnp.concatenate([f3l[...], f3h[...]], axis=1)

    wa0 = wat[0:_D, :]
    wa1 = wat[_D:2 * _D, :]
    b = batt[0, 0]

    def mv(f, w):
        return jnp.dot(f, w, preferred_element_type=f32)

    g0 = mv(f0, wa0)
    g1 = mv(f1, wa0)
    g2 = mv(f2, wa0)
    e0 = mv(f0, wa1)
    e1 = mv(f1, wa1)
    e2 = mv(f2, wa1)
    e3 = mv(f3, wa1)

    s0 = jnp.tanh(g0 + e0 + b)
    s1 = jnp.tanh(g0 + e1 + b)
    a = jax.nn.softmax(jnp.concatenate([s0, s1], axis=1), axis=1)
    s2 = jnp.tanh(a[:, 0:1] * g0 + a[:, 1:2] * g1 + e2 + b)
    a = jax.nn.softmax(jnp.concatenate([s0, s1, s2], axis=1), axis=1)
    s3 = jnp.tanh(a[:, 0:1] * g0 + a[:, 1:2] * g1 + a[:, 2:3] * g2 + e3 + b)

    a = jax.nn.softmax(jnp.concatenate([s0, s1, s2, s3], axis=1), axis=1)
    right = (f0 * a[:, 0:1] + f1 * a[:, 1:2]
             + f2 * a[:, 2:3] + f3 * a[:, 3:4])

    alpha = pa[0, 0]

    def prelu(v):
        return jnp.where(v >= 0, v, alpha * v)

    z = jnp.dot(right, w0[...], preferred_element_type=f32)
    h0 = z
    z = prelu(z)
    z = jnp.dot((1.0 - _ALPHA) * z + _ALPHA * h0, wg1[...],
                preferred_element_type=f32)
    z = prelu(z)
    z = jnp.dot((1.0 - _ALPHA) * z + _ALPHA * h0, wg2[...],
                preferred_element_type=f32)
    z = prelu(z)
    o_ref[...] = jnp.dot(z, w3[...], preferred_element_type=f32)


def _head(x, f1l, f1h, f2l, f2h, f3l, f3h, W_att, b_att,
          W0, Wg1, Wg2, W3, prelu_a):
    f32 = jnp.float32
    BN = 2000
    grid = (_N // BN,)

    def row_spec(w):
        return pl.BlockSpec((BN, w), lambda i: (i, 0))

    def full_spec(shape):
        return pl.BlockSpec(shape, lambda i: tuple(0 for _ in shape))

    return pl.pallas_call(
        _head_body,
        grid=grid,
        in_specs=[
            row_spec(_D),
            row_spec(_HALF), row_spec(_HALF),
            row_spec(_HALF), row_spec(_HALF),
            row_spec(_HALF), row_spec(_HALF),
            full_spec((2 * _D, 1)),
            full_spec((1, 1)),
            full_spec((_D, _HID)),
            full_spec((_HID, _HID)),
            full_spec((_HID, _HID)),
            full_spec((_HID, _D)),
            full_spec((1, 1)),
        ],
        out_specs=row_spec(_D),
        out_shape=jax.ShapeDtypeStruct((_N, _D), f32),
    )(x, f1l, f1h, f2l, f2h, f3l, f3h, W_att, b_att.reshape(1, 1),
      W0, Wg1, Wg2, W3, prelu_a.reshape(1, 1))


def kernel(x, edge_index, edge_weight, W_att, b_att, W0, Wg1, Wg2, W3,
           prelu_a):
    src2d = edge_index[0].astype(jnp.int32).reshape(_ROWS, 128)
    dst2d = edge_index[1].astype(jnp.int32).reshape(_ROWS, 128)
    ew2d = edge_weight.reshape(_ROWS, 128)
    idx3d = jnp.stack([src2d, dst2d], axis=1)

    x_lo = x[:, :_HALF]
    x_hi = x[:, _HALF:]

    f1l, f1h, f2l, f2h, f3l, f3h = _sc_layers(x_lo, x_hi, idx3d, ew2d)

    return _head(x, f1l, f1h, f2l, f2h, f3l, f3h,
                 W_att, b_att, W0, Wg1, Wg2, W3, prelu_a)
